# baseline (speedup 1.0000x reference)
import functools

import numpy as np
import jax
import jax.numpy as jnp
from jax import lax
from jax.experimental import pallas as pl
from jax.experimental.pallas import tpu as pltpu
from jax.experimental.pallas import tpu_sc as plsc

_NC = 2
_NS = 16
_NW = _NC * _NS
_LANES = 16


def _pe2_np(max_len, L, D):
    pos = np.arange(max_len)[:, None].astype(np.float32)
    div = np.exp(np.arange(0, D, 2).astype(np.float32) * (-np.log(10000.0) / D))
    pe = np.zeros((max_len, D), np.float32)
    pe[:, 0::2] = np.sin(pos * div)
    pe[:, 1::2] = np.cos(pos * div)
    pe = pe[:L]
    return pe + pe


@functools.partial(jax.jit, static_argnames=("L",))
def _sc_fused_lookup(pe2, seg, labels, L):
    N = labels.shape[0]
    S, D = seg.shape
    B = N // L
    n_w = N // _NW
    n_b = B // _NW
    CH = 128
    NBUF = 2
    n_ch = n_w // CH
    GRP = CH // _LANES
    GD = D // _LANES
    NBUILD = 15
    TB = (S * L) // NBUILD

    mesh = plsc.VectorSubcoreMesh(
        core_axis_name="c", subcore_axis_name="s",
        num_cores=_NC, num_subcores=_NS)

    @functools.partial(
        pl.kernel,
        out_type=jax.ShapeDtypeStruct((N, D), jnp.float32),
        mesh=mesh,
        scratch_types=[
            pltpu.VMEM_SHARED((S * L, D), jnp.float32),
            pltpu.VMEM((TB, D), jnp.float32),
            pltpu.VMEM((1, D), jnp.float32),
            pltpu.VMEM((n_w,), jnp.int32),
            pltpu.VMEM((n_ch, CH), jnp.int32),
        ] + [pltpu.VMEM((CH, D), jnp.float32) for _ in range(NBUF)]
          + [pltpu.SemaphoreType.DMA for _ in range(2 * NBUF + 1)],
    )
    def k(pe2_hbm, seg_hbm, labels_hbm, out_hbm,
          table_sh, bld_v, segrow_v, lab_v, idx_v, *bufs):
        rows = bufs[:NBUF]
        gs = bufs[NBUF:2 * NBUF]
        ws = bufs[2 * NBUF:3 * NBUF]
        lsem = bufs[3 * NBUF]
        sid = lax.axis_index("s")
        wid = sid * _NC + lax.axis_index("c")
        base = wid * n_w

        lab_copy = pltpu.make_async_copy(
            labels_hbm.at[pl.ds(base, n_w)], lab_v, lsem)
        lab_copy.start()

        @pl.when(sid < NBUILD)
        def _():
            r0 = sid * TB
            pltpu.sync_copy(pe2_hbm.at[pl.ds(lax.rem(r0, L), TB)], bld_v)
            pltpu.sync_copy(seg_hbm.at[pl.ds(r0 // L, 1)], segrow_v)
            for g in range(GD):
                sv = segrow_v[0, pl.ds(g * _LANES, _LANES)]
                for r in range(TB):
                    sl = pl.ds(g * _LANES, _LANES)
                    bld_v[r, sl] = bld_v[r, sl] + sv
            pltpu.sync_copy(bld_v, table_sh.at[pl.ds(r0, TB)])

        lab_copy.wait()
        plsc.subcore_barrier()

        iota = lax.iota(jnp.int32, _LANES)

        def compute_idx(c):
            for g in range(GRP):
                off = c * CH + g * _LANES
                lab = lab_v[pl.ds(off, _LANES)]
                p = base + off + iota
                idx_v[c, pl.ds(g * _LANES, _LANES)] = lab * L + lax.rem(p, L)

        def g_copy(c, b):
            return pltpu.make_async_copy(
                table_sh.at[idx_v.at[c]], rows[b], gs[b])

        def w_copy(c, b):
            return pltpu.make_async_copy(
                rows[b], out_hbm.at[pl.ds(base + c * CH, CH)], ws[b])

        for b in range(NBUF):
            compute_idx(b)
            g_copy(b, b).start()

        def body(i, carry):
            c = NBUF * i
            for b in range(NBUF):
                cc = c + b
                g_copy(cc, b).wait()
                w_copy(cc, b).start()
                compute_idx(cc + NBUF)
                w_copy(cc, b).wait()
                g_copy(cc + NBUF, b).start()
            return carry

        lax.fori_loop(0, (n_ch - NBUF) // NBUF, body, 0)

        for b in range(NBUF):
            cc = n_ch - NBUF + b
            g_copy(cc, b).wait()
            w_copy(cc, b).start()
            w_copy(cc, b).wait()

    return k(pe2, seg, labels)


def kernel(x, segment_label, seg_table):
    B, L = segment_label.shape
    S, D = seg_table.shape
    pe2 = jnp.asarray(_pe2_np(512, L, D))
    out = _sc_fused_lookup(pe2, seg_table, segment_label.reshape(B * L), L)
    return out.reshape(B, L, D)


# speedup vs baseline: 1.0127x; 1.0127x over previous
import functools

import numpy as np
import jax
import jax.numpy as jnp
from jax import lax
from jax.experimental import pallas as pl
from jax.experimental.pallas import tpu as pltpu
from jax.experimental.pallas import tpu_sc as plsc

_NC = 2
_NS = 16
_NW = _NC * _NS
_LANES = 16


def _pe2_np(max_len, L, D):
    pos = np.arange(max_len)[:, None].astype(np.float32)
    div = np.exp(np.arange(0, D, 2).astype(np.float32) * (-np.log(10000.0) / D))
    pe = np.zeros((max_len, D), np.float32)
    pe[:, 0::2] = np.sin(pos * div)
    pe[:, 1::2] = np.cos(pos * div)
    pe = pe[:L]
    return pe + pe


def _build_table(pe2, seg):
    S, D = seg.shape
    L = pe2.shape[0]

    def body(pe2_ref, seg_ref, out_ref):
        out_ref[...] = seg_ref[...][:, None, :] + pe2_ref[...][None, :, :]

    return pl.pallas_call(
        body,
        out_shape=jax.ShapeDtypeStruct((S, L, D), jnp.float32),
    )(pe2, seg)


@functools.partial(jax.jit, static_argnames=("L",))
def _sc_lookup(table, labels, L):
    N = labels.shape[0]
    D = table.shape[1]
    n_w = N // _NW
    CH = 128
    NBUF = 2
    n_ch = n_w // CH
    GRP = CH // _LANES

    mesh = plsc.VectorSubcoreMesh(
        core_axis_name="c", subcore_axis_name="s",
        num_cores=_NC, num_subcores=_NS)

    @functools.partial(
        pl.kernel,
        out_type=jax.ShapeDtypeStruct((N, D), jnp.float32),
        mesh=mesh,
        scratch_types=[
            pltpu.VMEM_SHARED(table.shape, jnp.float32),
            pltpu.VMEM((n_w,), jnp.int32),
            pltpu.VMEM((n_ch, CH), jnp.int32),
        ] + [pltpu.VMEM((CH, D), jnp.float32) for _ in range(NBUF)]
          + [pltpu.SemaphoreType.DMA for _ in range(2 * NBUF)],
    )
    def k(table_hbm, labels_hbm, out_hbm, table_sh, lab_v, idx_v, *bufs):
        rows = bufs[:NBUF]
        gs = bufs[NBUF:2 * NBUF]
        ws = bufs[2 * NBUF:3 * NBUF]
        sid = lax.axis_index("s")
        wid = sid * _NC + lax.axis_index("c")
        base = wid * n_w
        @pl.when(sid == 0)
        def _():
            pltpu.sync_copy(table_hbm, table_sh)
        pltpu.sync_copy(labels_hbm.at[pl.ds(base, n_w)], lab_v)
        plsc.subcore_barrier()
        iota = lax.iota(jnp.int32, _LANES)

        def compute_idx(c):
            for g in range(GRP):
                off = c * CH + g * _LANES
                lab = lab_v[pl.ds(off, _LANES)]
                p = base + off + iota
                idx_v[c, pl.ds(g * _LANES, _LANES)] = lab * L + lax.rem(p, L)

        def g_copy(c, b):
            return pltpu.make_async_copy(
                table_sh.at[idx_v.at[c]], rows[b], gs[b])

        def w_copy(c, b):
            return pltpu.make_async_copy(
                rows[b], out_hbm.at[pl.ds(base + c * CH, CH)], ws[b])

        for b in range(NBUF):
            compute_idx(b)
            g_copy(b, b).start()

        def body(i, carry):
            c = NBUF * i
            for b in range(NBUF):
                cc = c + b
                g_copy(cc, b).wait()
                w_copy(cc, b).start()
                compute_idx(cc + NBUF)
                w_copy(cc, b).wait()
                g_copy(cc + NBUF, b).start()
            return carry

        lax.fori_loop(0, (n_ch - NBUF) // NBUF, body, 0)

        for b in range(NBUF):
            cc = n_ch - NBUF + b
            g_copy(cc, b).wait()
            w_copy(cc, b).start()
            w_copy(cc, b).wait()

    return k(table, labels)


def kernel(x, segment_label, seg_table):
    B, L = segment_label.shape
    S, D = seg_table.shape
    pe2 = jnp.asarray(_pe2_np(512, L, D))
    table = _build_table(pe2, seg_table).reshape(S * L, D)
    labels = segment_label.reshape(B * L)
    out = _sc_lookup(table, labels, L)
    return out.reshape(B, L, D)
